# Initial kernel scaffold; baseline (speedup 1.0000x reference)
import jax
import jax.numpy as jnp
from jax.experimental import pallas as pl


def kernel(input, cloud_len_list, voxel_num, W1, W2, W3, W4, W5, g5, b5, Wl1, g6, b6, Wl2, bl2, g7, b7, Wl3, bl3, g3, b3, word_emb, Wl4, Wl5, bl5):
    raise NotImplementedError("write your pallas kernel here")


# speedup vs baseline: 7.1995x; 7.1995x over previous
import jax
import jax.numpy as jnp
from jax.experimental import pallas as pl
from jax.experimental.pallas import tpu as pltpu

P = 220
KNB = 10
NB = 128
EPS = 1e-5
NEG = -1e30
HI = jax.lax.Precision.HIGHEST


def _lrelu(x):
    return jnp.where(x > 0, x, 0.2 * x)


def _bfdot(a, b, dims):
    return jax.lax.dot_general(a.astype(jnp.bfloat16), b.astype(jnp.bfloat16),
                               (dims, ((), ())),
                               preferred_element_type=jnp.float32)


def _edge_layer(x, W, C):
    xx = jnp.sum(x * x, axis=0, keepdims=True)
    ip = _bfdot(x, x, ((0,), (0,)))
    inner = -2.0 * ip
    pd = ((-xx) - inner) - jnp.transpose(xx)
    sign = jnp.where(xx > 0, 1.0, 1e7)
    pd = pd * sign
    acc = None
    col = jax.lax.broadcasted_iota(jnp.int32, pd.shape, 1)
    for _ in range(KNB):
        rm = jnp.max(pd, axis=1, keepdims=True)
        first = jnp.min(jnp.where(pd == rm, col, P), axis=1, keepdims=True)
        oh = (col == first).astype(jnp.float32)
        pd = jnp.where(oh > 0, NEG, pd)
        feat = jax.lax.dot_general(x, oh, (((1,), (1,)), ((), ())),
                                   precision=HI,
                                   preferred_element_type=jnp.float32)
        f = jnp.concatenate([feat - x, x], axis=0)
        v = _bfdot(W, f, ((1,), (0,)))
        a = _lrelu(v)
        acc = a if acc is None else jnp.maximum(acc, a)
    return acc


def _stage_a(x_ref, w1_ref, w2_ref, w3_ref, w4_ref, w5_ref,
             cat_ref, stats_ref):
    x = x_ref[0]
    x1 = _edge_layer(x, w1_ref[...], 3)
    x2 = _edge_layer(x1, w2_ref[...], 32)
    x3 = _edge_layer(x2, w3_ref[...], 32)
    x4 = _edge_layer(x3, w4_ref[...], 64)
    cat = jnp.concatenate([x1, x2, x3, x4], axis=0)
    cat_ref[0] = cat
    y = _bfdot(w5_ref[...], cat, ((1,), (0,)))
    stats_ref[0, 0, :] = jnp.sum(y, axis=1)
    stats_ref[0, 1, :] = jnp.sum(y * y, axis=1)


def _stage_b(cat_ref, stats_ref, w5_ref, g5_ref, b5_ref, out_ref):
    stats = stats_ref[...]
    n = float(NB * P)
    s1 = jnp.sum(stats[:, 0, :], axis=0)
    s2 = jnp.sum(stats[:, 1, :], axis=0)
    m = s1 / n
    v = (s2 - 2.0 * m * s1 + n * m * m) / n
    cat = cat_ref[0]
    y = _bfdot(w5_ref[...], cat, ((1,), (0,)))
    yn = (y - m[:, None]) / jnp.sqrt(v + EPS)[:, None] * g5_ref[0][:, None] \
        + b5_ref[0][:, None]
    yn = _lrelu(yn)
    out_ref[0, 0, :] = jnp.max(yn, axis=1)
    out_ref[0, 1, :] = jnp.sum(yn, axis=1) / float(P)


def _bn_rows(x, g, b):
    m = jnp.sum(x, axis=0, keepdims=True) / float(NB)
    d = x - m
    v = jnp.sum(d * d, axis=0, keepdims=True) / float(NB)
    return (x - m) / jnp.sqrt(v + EPS) * g + b


def _stage_c(p_ref, wl1_ref, g6_ref, b6_ref, wl2_ref, bl2_ref, g7_ref,
             b7_ref, wl3_ref, bl3_ref, g3_ref, b3_ref, emb_ref, wl4_ref,
             wl5_ref, bl5_ref, out_ref):
    pool = p_ref[...]
    h = _bfdot(pool, wl1_ref[...], ((1,), (1,)))
    h = _lrelu(_bn_rows(h, g6_ref[...], b6_ref[...]))
    h = _bfdot(h, wl2_ref[...], ((1,), (1,))) + bl2_ref[...]
    h = _lrelu(_bn_rows(h, g7_ref[...], b7_ref[...]))
    h = _bfdot(h, wl3_ref[...], ((1,), (1,))) + bl3_ref[...]
    h = _lrelu(_bn_rows(h, g3_ref[...], b3_ref[...]))

    rm = jnp.max(h, axis=1, keepdims=True)
    col = jax.lax.broadcasted_iota(jnp.int32, h.shape, 1)
    first = jnp.min(jnp.where(h == rm, col, h.shape[1]), axis=1, keepdims=True)
    onehot = (col == first).astype(jnp.float32)

    rix = jax.lax.broadcasted_iota(jnp.int32, (4, NB), 1)
    dix = jax.lax.broadcasted_iota(jnp.int32, (4, NB), 0)
    seg = jnp.where(rix // 32 == dix, 1.0, 0.0)
    hist = jax.lax.dot_general(seg, onehot, (((1,), (0,)), ((), ())),
                               precision=HI,
                               preferred_element_type=jnp.float32)
    docs = jax.lax.dot_general(hist, emb_ref[...], (((1,), (0,)), ((), ())),
                               precision=HI,
                               preferred_element_type=jnp.float32) / 32.0
    d1 = _lrelu(_bfdot(docs, wl4_ref[...], ((1,), (1,))))
    out = _bfdot(d1, wl5_ref[...], ((1,), (1,))) + bl5_ref[...]
    out_ref[...] = out


def kernel(input, cloud_len_list, voxel_num, W1, W2, W3, W4, W5, g5, b5,
           Wl1, g6, b6, Wl2, bl2, g7, b7, Wl3, bl3, g3, b3, word_emb,
           Wl4, Wl5, bl5):
    B_doc, L = input.shape[0], input.shape[1]
    x = jnp.transpose(input.reshape(B_doc * L, P, 3), (0, 2, 1))

    cat_all, stats = pl.pallas_call(
        _stage_a,
        grid=(NB,),
        in_specs=[
            pl.BlockSpec((1, 3, P), lambda b: (b, 0, 0)),
            pl.BlockSpec(W1.shape, lambda b: (0, 0)),
            pl.BlockSpec(W2.shape, lambda b: (0, 0)),
            pl.BlockSpec(W3.shape, lambda b: (0, 0)),
            pl.BlockSpec(W4.shape, lambda b: (0, 0)),
            pl.BlockSpec(W5.shape, lambda b: (0, 0)),
        ],
        out_specs=[
            pl.BlockSpec((1, 256, P), lambda b: (b, 0, 0)),
            pl.BlockSpec((1, 2, 1024), lambda b: (b, 0, 0)),
        ],
        out_shape=[
            jax.ShapeDtypeStruct((NB, 256, P), jnp.float32),
            jax.ShapeDtypeStruct((NB, 2, 1024), jnp.float32),
        ],
        compiler_params=pltpu.CompilerParams(
            dimension_semantics=("parallel",)),
    )(x, W1, W2, W3, W4, W5)

    pooled = pl.pallas_call(
        _stage_b,
        grid=(NB,),
        in_specs=[
            pl.BlockSpec((1, 256, P), lambda b: (b, 0, 0)),
            pl.BlockSpec((NB, 2, 1024), lambda b: (0, 0, 0)),
            pl.BlockSpec(W5.shape, lambda b: (0, 0)),
            pl.BlockSpec((1, 1024), lambda b: (0, 0)),
            pl.BlockSpec((1, 1024), lambda b: (0, 0)),
        ],
        out_specs=pl.BlockSpec((1, 2, 1024), lambda b: (b, 0, 0)),
        out_shape=jax.ShapeDtypeStruct((NB, 2, 1024), jnp.float32),
        compiler_params=pltpu.CompilerParams(
            dimension_semantics=("parallel",)),
    )(cat_all, stats, W5, g5.reshape(1, -1), b5.reshape(1, -1))

    pooled2 = pooled.reshape(NB, 2048)

    out = pl.pallas_call(
        _stage_c,
        in_specs=[
            pl.BlockSpec((NB, 2048), lambda: (0, 0)),
            pl.BlockSpec(Wl1.shape, lambda: (0, 0)),
            pl.BlockSpec((1, 1024), lambda: (0, 0)),
            pl.BlockSpec((1, 1024), lambda: (0, 0)),
            pl.BlockSpec(Wl2.shape, lambda: (0, 0)),
            pl.BlockSpec((1, 1024), lambda: (0, 0)),
            pl.BlockSpec((1, 1024), lambda: (0, 0)),
            pl.BlockSpec((1, 1024), lambda: (0, 0)),
            pl.BlockSpec(Wl3.shape, lambda: (0, 0)),
            pl.BlockSpec((1, 759), lambda: (0, 0)),
            pl.BlockSpec((1, 759), lambda: (0, 0)),
            pl.BlockSpec((1, 759), lambda: (0, 0)),
            pl.BlockSpec(word_emb.shape, lambda: (0, 0)),
            pl.BlockSpec(Wl4.shape, lambda: (0, 0)),
            pl.BlockSpec(Wl5.shape, lambda: (0, 0)),
            pl.BlockSpec((1, 40), lambda: (0, 0)),
        ],
        out_specs=pl.BlockSpec((4, 40), lambda: (0, 0)),
        out_shape=jax.ShapeDtypeStruct((4, 40), jnp.float32),
    )(pooled2, Wl1, g6.reshape(1, -1), b6.reshape(1, -1), Wl2,
      bl2.reshape(1, -1), g7.reshape(1, -1), b7.reshape(1, -1), Wl3,
      bl3.reshape(1, -1), g3.reshape(1, -1), b3.reshape(1, -1), word_emb,
      Wl4, Wl5, bl5.reshape(1, -1))
    return out
